# baseline (speedup 1.0000x reference)
import functools

import jax
import jax.numpy as jnp
from jax import lax
from jax.experimental import pallas as pl
from jax.experimental.pallas import tpu as pltpu
from jax.experimental.pallas import tpu_sc as plsc

_M = 0.2
_S = 30.0
_LOG2E = 1.4426950408889634
_K1 = _S * _LOG2E

_W = 1024
_G = 7


def _sc_gather_targets(inputs, targets):
    b, c = inputs.shape
    info = plsc.get_sparse_core_info()
    nw = info.num_cores * info.num_subcores
    bpw = b // nw
    mesh = plsc.VectorSubcoreMesh(core_axis_name="c", subcore_axis_name="s")

    @functools.partial(
        pl.kernel,
        mesh=mesh,
        out_type=jax.ShapeDtypeStruct((b,), jnp.float32),
        scratch_types=[
            pltpu.VMEM((bpw,), jnp.int32),
            pltpu.VMEM((bpw, 8, 128), jnp.float32),
            pltpu.VMEM((bpw,), jnp.float32),
            pltpu.SemaphoreType.DMA,
        ],
    )
    def gather(in_hbm, tgt_hbm, out_hbm, idx_v, tiles_v, xts_v, sem):
        wid = lax.axis_index("s") * info.num_cores + lax.axis_index("c")
        base = pl.multiple_of(wid * bpw, bpw)
        pltpu.sync_copy(tgt_hbm.at[pl.ds(base, bpw)], idx_v)
        lanes = lax.iota(jnp.int32, 16)
        ts, handles = [], []
        for chunk in range(bpw // 16):
            tv = idx_v[pl.ds(chunk * 16, 16)]
            for l in range(16):
                k = chunk * 16 + l
                t = tv[l]
                ts.append(t)
                cb = pl.multiple_of(jnp.bitwise_and(t, jnp.int32(-128)), 128)
                rb = (k // 8) * 8
                handles.append(
                    pltpu.async_copy(
                        in_hbm.at[pl.ds(base + rb, 8), pl.ds(cb, 128)],
                        tiles_v.at[k], sem))
        for h in handles:
            h.wait()
        for chunk in range(bpw // 16):
            xt_acc = jnp.zeros((16,), jnp.float32)
            for l in range(16):
                k = chunk * 16 + l
                lane = jnp.bitwise_and(ts[k], 127)
                sel = jnp.zeros((16,), jnp.float32)
                for l8 in range(8):
                    v = tiles_v[k, k % 8, pl.ds(l8 * 16, 16)]
                    sel = jnp.where(l8 * 16 + lanes == lane, v, sel)
                idxv = jnp.full((16,), jnp.bitwise_and(lane, 15), jnp.int32)
                v16 = lax.gather(
                    sel, idxv[:, None],
                    lax.GatherDimensionNumbers(
                        offset_dims=(), collapsed_slice_dims=(0,),
                        start_index_map=(0,)),
                    slice_sizes=(1,),
                    mode=lax.GatherScatterMode.PROMISE_IN_BOUNDS)
                xt_acc = jnp.where(lanes == l, v16, xt_acc)
            xts_v[pl.ds(chunk * 16, 16)] = xt_acc
        pltpu.sync_copy(xts_v, out_hbm.at[pl.ds(base, bpw)])

    return gather(inputs, targets)


def _tc_body(ng, c, *refs):
    x_refs = refs[:_G]
    xt_ref, o_ref, acc = refs[_G:]
    nc = ng * _G
    j = pl.program_id(0)

    @pl.when(j == 0)
    def _():
        acc[...] = jnp.zeros_like(acc)

    @pl.when(j < ng - 1)
    def _():
        s = jnp.zeros_like(acc)
        for g in range(_G):
            e = jnp.exp2(x_refs[g][...] * _K1)
            s += jnp.sum(e, axis=1, keepdims=True)
        acc[...] += s

    @pl.when(j == ng - 1)
    def _():
        s = acc[...]
        for g in range(_G - 1):
            e = jnp.exp2(x_refs[g][...] * _K1)
            s += jnp.sum(e, axis=1, keepdims=True)
        cols = (nc - 1) * _W + jax.lax.broadcasted_iota(jnp.int32, (1, _W), 1)
        e = jnp.exp2(x_refs[_G - 1][...] * _K1)
        e = jnp.where(cols < c, e, 0.0)
        s += jnp.sum(e, axis=1, keepdims=True)
        xt = xt_ref[...]
        e_old = jnp.exp2(xt * _K1)
        e_new = jnp.exp2((xt - _M) * _K1)
        s_mod = s - e_old + e_new
        loss = jnp.log(s_mod) - _S * (xt - _M)
        o_ref[...] = jnp.mean(loss, keepdims=True)


def kernel(inputs, targets):
    b, c = inputs.shape
    nc = pl.cdiv(c, _W)
    ng = nc // _G
    xt = _sc_gather_targets(inputs, targets).reshape(b, 1)
    in_specs = [
        pl.BlockSpec((b, _W), functools.partial(
            lambda g, j: (0, g * ng + j), g))
        for g in range(_G)
    ]
    in_specs.append(pl.BlockSpec((b, 1), lambda j: (0, 0)))
    out = pl.pallas_call(
        functools.partial(_tc_body, ng, c),
        grid=(ng,),
        in_specs=in_specs,
        out_specs=pl.BlockSpec((1, 1), lambda j: (0, 0)),
        out_shape=jax.ShapeDtypeStruct((1, 1), jnp.float32),
        scratch_shapes=[pltpu.VMEM((b, 1), jnp.float32)],
        compiler_params=pltpu.CompilerParams(
            vmem_limit_bytes=100 * 1024 * 1024),
    )(*([inputs] * _G), xt)
    return out[0, 0]


# speedup vs baseline: 1.0394x; 1.0394x over previous
import functools

import jax
import jax.numpy as jnp
from jax.experimental import pallas as pl
from jax.experimental.pallas import tpu as pltpu

_M = 0.2
_S = 30.0
_LOG2E = 1.4426950408889634
_K1 = _S * _LOG2E

_W = 512
_G = 7


def _tc_body(ng, c, *refs):
    x_refs = refs[:_G]
    t_ref, o_ref, acc, tacc = refs[_G:]
    nc = ng * _G
    j = pl.program_id(0)

    @pl.when(j == 0)
    def _():
        acc[...] = jnp.zeros_like(acc)
        tacc[...] = jnp.zeros_like(tacc)

    t = t_ref[...]
    iota = jax.lax.broadcasted_iota(jnp.int32, (1, _W), 1)

    @pl.when(j < ng - 1)
    def _():
        s = jnp.zeros_like(acc)
        xt = jnp.zeros_like(tacc)
        for g in range(_G):
            x = x_refs[g][...]
            e = jnp.exp2(x * _K1)
            s += jnp.sum(e, axis=1, keepdims=True)
            cols = (g * ng + j) * _W + iota
            xt += jnp.sum(jnp.where(cols == t, x, 0.0), axis=1,
                          keepdims=True)
        acc[...] += s
        tacc[...] += xt

    @pl.when(j == ng - 1)
    def _():
        s = acc[...]
        xt = tacc[...]
        for g in range(_G):
            x = x_refs[g][...]
            cols = (g * ng + j) * _W + iota
            e = jnp.exp2(x * _K1)
            if g == _G - 1:
                e = jnp.where(cols < c, e, 0.0)
            s += jnp.sum(e, axis=1, keepdims=True)
            xt += jnp.sum(jnp.where(cols == t, x, 0.0), axis=1,
                          keepdims=True)
        e_old = jnp.exp2(xt * _K1)
        e_new = jnp.exp2((xt - _M) * _K1)
        s_mod = s - e_old + e_new
        loss = jnp.log(s_mod) - _S * (xt - _M)
        o_ref[...] = jnp.mean(loss, keepdims=True)


def kernel(inputs, targets):
    b, c = inputs.shape
    nc = pl.cdiv(c, _W)
    ng = nc // _G
    t2 = targets.reshape(b, 1)
    in_specs = [
        pl.BlockSpec((b, _W), functools.partial(
            lambda g, j: (0, g * ng + j), g))
        for g in range(_G)
    ]
    in_specs.append(pl.BlockSpec((b, 1), lambda j: (0, 0)))
    out = pl.pallas_call(
        functools.partial(_tc_body, ng, c),
        grid=(ng,),
        in_specs=in_specs,
        out_specs=pl.BlockSpec((1, 1), lambda j: (0, 0)),
        out_shape=jax.ShapeDtypeStruct((1, 1), jnp.float32),
        scratch_shapes=[
            pltpu.VMEM((b, 1), jnp.float32),
            pltpu.VMEM((b, 1), jnp.float32),
        ],
        compiler_params=pltpu.CompilerParams(
            vmem_limit_bytes=100 * 1024 * 1024),
    )(*([inputs] * _G), t2)
    return out[0, 0]
